# baseline (speedup 1.0000x reference)
import functools

import jax
import jax.numpy as jnp
from jax import lax
from jax.experimental import pallas as pl
from jax.experimental.pallas import tpu as pltpu
from jax.experimental.pallas import tpu_sc as plsc

N, E, T, NORB = 10000, 160000, 320000, 9
HD, CD, VD = 128, 16, 32

BN = 2000
BE_C = 1000
BE_P = 2000


def _silu(v):
    return v * jax.nn.sigmoid(v)


def _full(shape):
    return pl.BlockSpec(shape, lambda i: (0,) * len(shape))


def _prenode_body(x_ref, wn_ref, bn_ref, x1_ref, xks_ref):
    h = jnp.dot(x_ref[...], wn_ref[...], preferred_element_type=jnp.float32)
    h = h + bn_ref[...]
    x1_ref[...] = h[:, :VD]
    xks_ref[...] = jax.nn.sigmoid(h[:, VD:])


def _prenode(x, W_node, b_node):
    return pl.pallas_call(
        _prenode_body,
        grid=(N // BN,),
        in_specs=[
            pl.BlockSpec((BN, HD), lambda i: (i, 0)),
            _full((HD, 2 * VD)),
            _full((1, 2 * VD)),
        ],
        out_specs=[
            pl.BlockSpec((BN, VD), lambda i: (i, 0)),
            pl.BlockSpec((BN, VD), lambda i: (i, 0)),
        ],
        out_shape=[
            jax.ShapeDtypeStruct((N, VD), jnp.float32),
            jax.ShapeDtypeStruct((N, VD), jnp.float32),
        ],
    )(x, W_node, b_node.reshape(1, 2 * VD))


def _cmlp_body(cji_ref, rb_ref, cut_ref, wc1_ref, wc2_ref, cw_ref, s_ref):
    w1 = wc1_ref[...]
    w2 = wc2_ref[...]
    cut = cut_ref[...]
    s = jnp.zeros((BE_C, VD), jnp.float32)
    for d in range(NORB):
        cd = _silu(jnp.dot(_silu(jnp.dot(cji_ref[:, d, :], w1,
                                         preferred_element_type=jnp.float32)),
                           w2, preferred_element_type=jnp.float32))
        cwd = cd * (rb_ref[:, d:d + 1] * cut)
        cw_ref[:, d * VD:(d + 1) * VD] = cwd
        s = s + cwd
    s_ref[...] = s


def _cmlp(cji, rb, cutoff_w, W_c1, W_c2):
    return pl.pallas_call(
        _cmlp_body,
        grid=(E // BE_C,),
        in_specs=[
            pl.BlockSpec((BE_C, NORB, CD), lambda i: (i, 0, 0)),
            pl.BlockSpec((BE_C, NORB), lambda i: (i, 0)),
            pl.BlockSpec((BE_C, 1), lambda i: (i, 0)),
            _full((CD, VD)),
            _full((VD, VD)),
        ],
        out_specs=[
            pl.BlockSpec((BE_C, NORB * VD), lambda i: (i, 0)),
            pl.BlockSpec((BE_C, VD), lambda i: (i, 0)),
        ],
        out_shape=[
            jax.ShapeDtypeStruct((E, NORB * VD), jnp.float32),
            jax.ShapeDtypeStruct((E, VD), jnp.float32),
        ],
    )(cji, rb, cutoff_w.reshape(E, 1), W_c1, W_c2)


def _post_body(agg_ref, s_ref, nfa_ref, nfb_ref, wt1_ref, bt1_ref, wt2_ref,
               bt2_ref, wb_ref, wn1a_ref, wn1b_ref, bn1_ref, wn2_ref,
               bn2_ref, msg_ref):
    tbw = _silu(jnp.dot(_silu(jnp.dot(agg_ref[...], wt1_ref[...],
                                      preferred_element_type=jnp.float32)
                              + bt1_ref[...]),
                        wt2_ref[...], preferred_element_type=jnp.float32)
                + bt2_ref[...])
    lcao = (1.0 + tbw) * s_ref[...]
    n2 = jnp.sum(lcao * lcao, axis=-1, keepdims=True)
    lcao = lcao * jax.lax.rsqrt(jnp.maximum(n2, 1e-24))
    lcao = jnp.dot(lcao, wb_ref[...], preferred_element_type=jnp.float32)
    nf1 = (jnp.dot(nfa_ref[...], wn1a_ref[...],
                   preferred_element_type=jnp.float32)
           + jnp.dot(nfb_ref[...], wn1b_ref[...],
                     preferred_element_type=jnp.float32)
           + bn1_ref[...])
    nf = _silu(jnp.dot(_silu(nf1), wn2_ref[...],
                       preferred_element_type=jnp.float32)
               + bn2_ref[...])
    msg_ref[...] = lcao * nf


def _post(agg_e, s, nf_a, nf_b, W_t1, b_t1, W_t2, b_t2, W_basis,
          W_n1, b_n1, W_n2, b_n2):
    return pl.pallas_call(
        _post_body,
        grid=(E // BE_P,),
        in_specs=[
            pl.BlockSpec((BE_P, VD), lambda i: (i, 0)),
            pl.BlockSpec((BE_P, VD), lambda i: (i, 0)),
            pl.BlockSpec((BE_P, VD), lambda i: (i, 0)),
            pl.BlockSpec((BE_P, VD), lambda i: (i, 0)),
            _full((VD, VD)), _full((1, VD)),
            _full((VD, VD)), _full((1, VD)),
            _full((VD, VD)),
            _full((VD, VD)), _full((VD, VD)), _full((1, VD)),
            _full((VD, VD)), _full((1, VD)),
        ],
        out_specs=pl.BlockSpec((BE_P, VD), lambda i: (i, 0)),
        out_shape=jax.ShapeDtypeStruct((E, VD), jnp.float32),
    )(agg_e, s, nf_a, nf_b, W_t1, b_t1.reshape(1, VD), W_t2,
      b_t2.reshape(1, VD), W_basis, W_n1[:VD], W_n1[VD:],
      b_n1.reshape(1, VD), W_n2, b_n2.reshape(1, VD))


def _final_body(x_ref, agg_ref, wo_ref, out_ref):
    agg = agg_ref[0] + agg_ref[1]
    out_ref[...] = x_ref[...] + jnp.dot(agg, wo_ref[...],
                                        preferred_element_type=jnp.float32)


def _final(x, agg_n2, W_out):
    return pl.pallas_call(
        _final_body,
        grid=(N // BN,),
        in_specs=[
            pl.BlockSpec((BN, HD), lambda i: (i, 0)),
            pl.BlockSpec((2, BN, VD), lambda i: (0, i, 0)),
            _full((VD, HD)),
        ],
        out_specs=pl.BlockSpec((BN, HD), lambda i: (i, 0)),
        out_shape=jax.ShapeDtypeStruct((N, HD), jnp.float32),
    )(x, agg_n2, W_out)


NW = 32
TPW = T // NW
BT = 200
NIT = TPW // BT


def _rsqrt_scalar(x):
    i = lax.bitcast_convert_type(x, jnp.int32)
    i = jnp.int32(0x5F3759DF) - (i >> 1)
    y = lax.bitcast_convert_type(i, jnp.float32)
    for _ in range(3):
        y = y * (1.5 - 0.5 * x * y * y)
    return y


def _tri_body(cw_hbm, shb_hbm, ekj_hbm, tk_hbm, xks_hbm, out_hbm,
              ekj_v, tk_v, rows_v, shb_v, xk_v, out_v, sem1, sem2):
    wid = lax.axis_index("s") * 2 + lax.axis_index("c")
    base = wid * TPW
    pltpu.sync_copy(ekj_hbm.at[pl.ds(base, TPW)], ekj_v)
    pltpu.sync_copy(tk_hbm.at[pl.ds(base, TPW)], tk_v)

    def body(it, carry):
        off = base + it * BT
        loc = it * BT
        pltpu.sync_copy(shb_hbm.at[pl.ds(off * NORB, BT * NORB)],
                        shb_v.at[pl.ds(0, BT * NORB)])
        cp1 = pltpu.async_copy(cw_hbm.at[ekj_v.at[pl.ds(loc, BT)]], rows_v,
                               sem1)
        cp2 = pltpu.async_copy(xks_hbm.at[tk_v.at[pl.ds(loc, BT)]], xk_v,
                               sem2)
        cp1.wait()
        cp2.wait()

        def per_t(t, tcarry):
            acc0 = jnp.zeros((16,), jnp.float32)
            acc1 = jnp.zeros((16,), jnp.float32)
            shrow = shb_v[pl.ds(t * NORB, 16)]
            for d in range(NORB):
                sh = shrow[d]
                acc0 = acc0 + sh * rows_v[t, pl.ds(d * VD, 16)]
                acc1 = acc1 + sh * rows_v[t, pl.ds(d * VD + 16, 16)]
            nsq = jnp.sum(acc0 * acc0 + acc1 * acc1, axis=0)
            rinv = _rsqrt_scalar(jnp.maximum(nsq, 1e-24))
            out_v[t, pl.ds(0, 16)] = acc0 * rinv * xk_v[t, pl.ds(0, 16)]
            out_v[t, pl.ds(16, 16)] = acc1 * rinv * xk_v[t, pl.ds(16, 16)]
            return tcarry

        lax.fori_loop(0, BT, per_t, 0, unroll=2)
        pltpu.sync_copy(out_v, out_hbm.at[pl.ds(off, BT)])
        return carry

    lax.fori_loop(0, NIT, body, 0)


def _tri_sc(cw, shb, edge_idx_kj, tri_idx_k, xks):
    mesh = plsc.VectorSubcoreMesh(core_axis_name="c", subcore_axis_name="s")
    f = pl.kernel(
        _tri_body,
        out_type=jax.ShapeDtypeStruct((T, VD), jnp.float32),
        mesh=mesh,
        compiler_params=pltpu.CompilerParams(needs_layout_passes=False,
                                             use_tc_tiling_on_sc=False),
        scratch_types=[
            pltpu.VMEM((TPW,), jnp.int32),
            pltpu.VMEM((TPW,), jnp.int32),
            pltpu.VMEM((BT, NORB * VD), jnp.float32),
            pltpu.VMEM((BT * NORB + 16,), jnp.float32),
            pltpu.VMEM((BT, VD), jnp.float32),
            pltpu.VMEM((BT, VD), jnp.float32),
            pltpu.SemaphoreType.DMA,
            pltpu.SemaphoreType.DMA,
        ],
    )
    return f(cw, shb, edge_idx_kj, tri_idx_k, xks)


ECH = 40000
NDUMP = 64
BV = 400
TPT = T // 16
ZROWS = (ECH + NDUMP) // 16
WROWS = ECH // 16


def _seg_e_body(tbw_hbm, eji_hbm, agg_hbm, idx_v, idx2_v, vals_v, zbuf_v,
                acc_sh, sem1):
    c = lax.axis_index("c")
    s = lax.axis_index("s")
    zero16 = jnp.zeros((16,), jnp.float32)

    def zrow(r, carry):
        zbuf_v[r, pl.ds(0, 16)] = zero16
        zbuf_v[r, pl.ds(16, 16)] = zero16
        return carry

    lax.fori_loop(0, BV, zrow, 0)

    for chunk_i in range(2):
        eb = (c * 2 + chunk_i) * ECH
        zb = s * ZROWS
        for j in range(ZROWS // BV):
            pltpu.sync_copy(zbuf_v, acc_sh.at[pl.ds(zb + j * BV, BV)])
        rem = ZROWS % BV
        pltpu.sync_copy(zbuf_v.at[pl.ds(0, rem)],
                        acc_sh.at[pl.ds(zb + (ZROWS // BV) * BV, rem)])
        plsc.subcore_barrier()

        tbase = s * TPT

        def rnd(r, carry):
            off = tbase + r * BV
            pltpu.sync_copy(eji_hbm.at[pl.ds(off, BV)], idx_v)
            pltpu.sync_copy(tbw_hbm.at[pl.ds(off, BV)], vals_v)

            def ix(j, jcarry):
                raw = idx_v[pl.ds(j * 16, 16)]
                v = raw - eb
                ok = (v >= 0) & (v < ECH)
                dump = ECH + (raw & (NDUMP - 1))
                idx2_v[pl.ds(j * 16, 16)] = jnp.where(ok, v, dump)
                return jcarry

            lax.fori_loop(0, BV // 16, ix, 0)
            pltpu.sync_copy(vals_v, acc_sh.at[idx2_v], add=True)
            return carry

        lax.fori_loop(0, TPT // BV, rnd, 0)
        plsc.subcore_barrier()

        wb = s * WROWS
        for j in range(WROWS // BV):
            pltpu.sync_copy(acc_sh.at[pl.ds(wb + j * BV, BV)], vals_v)
            pltpu.sync_copy(vals_v, agg_hbm.at[pl.ds(eb + wb + j * BV, BV)])
        remw = WROWS % BV
        pltpu.sync_copy(acc_sh.at[pl.ds(wb + (WROWS // BV) * BV, remw)],
                        vals_v.at[pl.ds(0, remw)])
        pltpu.sync_copy(vals_v.at[pl.ds(0, remw)],
                        agg_hbm.at[pl.ds(eb + wb + (WROWS // BV) * BV, remw)])
        plsc.subcore_barrier()


def _seg_e_sc(tbw_t, edge_idx_ji):
    mesh = plsc.VectorSubcoreMesh(core_axis_name="c", subcore_axis_name="s")
    f = pl.kernel(
        _seg_e_body,
        out_type=jax.ShapeDtypeStruct((E, VD), jnp.float32),
        mesh=mesh,
        compiler_params=pltpu.CompilerParams(needs_layout_passes=False,
                                             use_tc_tiling_on_sc=False),
        scratch_types=[
            pltpu.VMEM((BV,), jnp.int32),
            pltpu.VMEM((BV,), jnp.int32),
            pltpu.VMEM((BV, VD), jnp.float32),
            pltpu.VMEM((BV, VD), jnp.float32),
            pltpu.VMEM_SHARED((ECH + NDUMP, VD), jnp.float32),
            pltpu.SemaphoreType.DMA,
        ],
    )
    return f(tbw_t, edge_idx_ji)


EPW = E // NW
BG = 200


def _pair_body(x1_hbm, ii_hbm, jj_hbm, nfa_hbm, nfb_hbm,
               ii_v, jj_v, ra_v, rb_v, sem1, sem2):
    wid = lax.axis_index("s") * 2 + lax.axis_index("c")
    base = wid * EPW

    def rnd(r, carry):
        off = base + r * BG
        pltpu.sync_copy(ii_hbm.at[pl.ds(off, BG)], ii_v)
        pltpu.sync_copy(jj_hbm.at[pl.ds(off, BG)], jj_v)
        cp1 = pltpu.async_copy(x1_hbm.at[ii_v], ra_v, sem1)
        cp2 = pltpu.async_copy(x1_hbm.at[jj_v], rb_v, sem2)
        cp1.wait()
        cp2.wait()
        pltpu.sync_copy(ra_v, nfa_hbm.at[pl.ds(off, BG)])
        pltpu.sync_copy(rb_v, nfb_hbm.at[pl.ds(off, BG)])
        return carry

    lax.fori_loop(0, EPW // BG, rnd, 0)


def _pair_sc(x1, idx_i, idx_j):
    mesh = plsc.VectorSubcoreMesh(core_axis_name="c", subcore_axis_name="s")
    f = pl.kernel(
        _pair_body,
        out_type=[jax.ShapeDtypeStruct((E, VD), jnp.float32),
                  jax.ShapeDtypeStruct((E, VD), jnp.float32)],
        mesh=mesh,
        compiler_params=pltpu.CompilerParams(needs_layout_passes=False,
                                             use_tc_tiling_on_sc=False),
        scratch_types=[
            pltpu.VMEM((BG,), jnp.int32),
            pltpu.VMEM((BG,), jnp.int32),
            pltpu.VMEM((BG, VD), jnp.float32),
            pltpu.VMEM((BG, VD), jnp.float32),
            pltpu.SemaphoreType.DMA,
            pltpu.SemaphoreType.DMA,
        ],
    )
    return f(x1, idx_i, idx_j)


EPC = E // 2
EPT = EPC // 16
NZR = N // 16
BVN = 200


def _seg_n_body(msg_hbm, ii_hbm, agg_hbm, idx_v, vals_v, zbuf_v, acc_sh,
                sem1):
    c = lax.axis_index("c")
    s = lax.axis_index("s")
    zero16 = jnp.zeros((16,), jnp.float32)

    def zrow(r, carry):
        zbuf_v[r, pl.ds(0, 16)] = zero16
        zbuf_v[r, pl.ds(16, 16)] = zero16
        return carry

    lax.fori_loop(0, BVN, zrow, 0)
    zb = s * NZR
    for j in range(NZR // BVN):
        pltpu.sync_copy(zbuf_v, acc_sh.at[pl.ds(zb + j * BVN, BVN)])
    remz = NZR % BVN
    pltpu.sync_copy(zbuf_v.at[pl.ds(0, remz)],
                    acc_sh.at[pl.ds(zb + NZR - remz, remz)])
    plsc.subcore_barrier()

    tbase = c * EPC + s * EPT

    def rnd(r, carry):
        off = tbase + r * BVN
        pltpu.sync_copy(ii_hbm.at[pl.ds(off, BVN)], idx_v)
        pltpu.sync_copy(msg_hbm.at[pl.ds(off, BVN)], vals_v)
        pltpu.sync_copy(vals_v, acc_sh.at[idx_v], add=True)
        return carry

    lax.fori_loop(0, EPT // BVN, rnd, 0)
    plsc.subcore_barrier()

    for j in range(NZR // BVN):
        pltpu.sync_copy(acc_sh.at[pl.ds(zb + j * BVN, BVN)], vals_v)
        pltpu.sync_copy(vals_v, agg_hbm.at[c, pl.ds(zb + j * BVN, BVN)])
    pltpu.sync_copy(acc_sh.at[pl.ds(zb + NZR - remz, remz)],
                    vals_v.at[pl.ds(0, remz)])
    pltpu.sync_copy(vals_v.at[pl.ds(0, remz)],
                    agg_hbm.at[c, pl.ds(zb + NZR - remz, remz)])


def _seg_n_sc(msg, idx_i):
    mesh = plsc.VectorSubcoreMesh(core_axis_name="c", subcore_axis_name="s")
    f = pl.kernel(
        _seg_n_body,
        out_type=jax.ShapeDtypeStruct((2, N, VD), jnp.float32),
        mesh=mesh,
        compiler_params=pltpu.CompilerParams(needs_layout_passes=False,
                                             use_tc_tiling_on_sc=False),
        scratch_types=[
            pltpu.VMEM((BVN,), jnp.int32),
            pltpu.VMEM((BVN, VD), jnp.float32),
            pltpu.VMEM((BVN, VD), jnp.float32),
            pltpu.VMEM_SHARED((N, VD), jnp.float32),
            pltpu.SemaphoreType.DMA,
        ],
    )
    return f(msg, idx_i)


def kernel(x, cji, valence_mask, cutoff_w, rb, shb, idx_i, idx_j, tri_idx_k,
           edge_idx_kj, edge_idx_ji, W_node, b_node, W_c1, W_c2, W_t1, b_t1,
           W_t2, b_t2, W_basis, W_n1, b_n1, W_n2, b_n2, W_out):
    x1, xks = _prenode(x, W_node, b_node)
    cw, s = _cmlp(cji, rb, cutoff_w, W_c1, W_c2)

    tbw_t = _tri_sc(cw, shb.reshape(T * NORB), edge_idx_kj, tri_idx_k, xks)
    agg_e = _seg_e_sc(tbw_t, edge_idx_ji)

    nf_a, nf_b = _pair_sc(x1, idx_i, idx_j)

    msg = _post(agg_e, s, nf_a, nf_b, W_t1, b_t1, W_t2, b_t2, W_basis,
                W_n1, b_n1, W_n2, b_n2)

    agg_n2 = _seg_n_sc(msg, idx_i)
    return _final(x, agg_n2, W_out)


# speedup vs baseline: 1.3845x; 1.3845x over previous
import functools

import jax
import jax.numpy as jnp
from jax import lax
from jax.experimental import pallas as pl
from jax.experimental.pallas import tpu as pltpu
from jax.experimental.pallas import tpu_sc as plsc

N, E, T, NORB = 10000, 160000, 320000, 9
HD, CD, VD = 128, 16, 32

BN = 2000
BE_C = 1000
BE_P = 2000


def _silu(v):
    return v * jax.nn.sigmoid(v)


def _full(shape):
    return pl.BlockSpec(shape, lambda i: (0,) * len(shape))


def _prenode_body(x_ref, wn_ref, bn_ref, x1_ref, xks_ref):
    h = jnp.dot(x_ref[...], wn_ref[...], preferred_element_type=jnp.float32)
    h = h + bn_ref[...]
    x1_ref[...] = h[:, :VD]
    xks_ref[...] = jax.nn.sigmoid(h[:, VD:])


def _prenode(x, W_node, b_node):
    return pl.pallas_call(
        _prenode_body,
        grid=(N // BN,),
        in_specs=[
            pl.BlockSpec((BN, HD), lambda i: (i, 0)),
            _full((HD, 2 * VD)),
            _full((1, 2 * VD)),
        ],
        out_specs=[
            pl.BlockSpec((BN, VD), lambda i: (i, 0)),
            pl.BlockSpec((BN, VD), lambda i: (i, 0)),
        ],
        out_shape=[
            jax.ShapeDtypeStruct((N, VD), jnp.float32),
            jax.ShapeDtypeStruct((N, VD), jnp.float32),
        ],
    )(x, W_node, b_node.reshape(1, 2 * VD))


def _cmlp_body(cji_ref, rb_ref, cut_ref, w1_ref, w2_ref, r_ref, sm_ref,
               cw_ref, s_ref):
    z = _silu(jnp.dot(cji_ref[...], w1_ref[...],
                      preferred_element_type=jnp.float32))
    z = _silu(jnp.dot(z, w2_ref[...], preferred_element_type=jnp.float32))
    rbw = jnp.dot(rb_ref[...] * cut_ref[...], r_ref[...],
                  preferred_element_type=jnp.float32)
    cw = z * rbw
    cw_ref[...] = cw
    s_ref[...] = jnp.dot(cw, sm_ref[...], preferred_element_type=jnp.float32)


def _cmlp(cji, rb, cutoff_w, W_c1, W_c2):
    z16 = jnp.zeros((CD, VD), jnp.float32)
    z32 = jnp.zeros((VD, VD), jnp.float32)
    w1bd = jnp.concatenate(
        [jnp.concatenate([W_c1 if i == d else z16 for i in range(NORB)],
                         axis=1) for d in range(NORB)], axis=0)
    w2bd = jnp.concatenate(
        [jnp.concatenate([W_c2 if i == d else z32 for i in range(NORB)],
                         axis=1) for d in range(NORB)], axis=0)
    rmap = jnp.repeat(jnp.eye(NORB, dtype=jnp.float32), VD, axis=1)
    smap = jnp.tile(jnp.eye(VD, dtype=jnp.float32), (NORB, 1))
    return pl.pallas_call(
        _cmlp_body,
        grid=(E // BE_C,),
        in_specs=[
            pl.BlockSpec((BE_C, NORB * CD), lambda i: (i, 0)),
            pl.BlockSpec((BE_C, NORB), lambda i: (i, 0)),
            pl.BlockSpec((BE_C, 1), lambda i: (i, 0)),
            _full((NORB * CD, NORB * VD)),
            _full((NORB * VD, NORB * VD)),
            _full((NORB, NORB * VD)),
            _full((NORB * VD, VD)),
        ],
        out_specs=[
            pl.BlockSpec((BE_C, NORB * VD), lambda i: (i, 0)),
            pl.BlockSpec((BE_C, VD), lambda i: (i, 0)),
        ],
        out_shape=[
            jax.ShapeDtypeStruct((E, NORB * VD), jnp.float32),
            jax.ShapeDtypeStruct((E, VD), jnp.float32),
        ],
    )(cji.reshape(E, NORB * CD), rb, cutoff_w.reshape(E, 1),
      w1bd, w2bd, rmap, smap)


def _post_body(agg_ref, s_ref, nfa_ref, nfb_ref, wt1_ref, bt1_ref, wt2_ref,
               bt2_ref, wb_ref, wn1a_ref, wn1b_ref, bn1_ref, wn2_ref,
               bn2_ref, msg_ref):
    tbw = _silu(jnp.dot(_silu(jnp.dot(agg_ref[...], wt1_ref[...],
                                      preferred_element_type=jnp.float32)
                              + bt1_ref[...]),
                        wt2_ref[...], preferred_element_type=jnp.float32)
                + bt2_ref[...])
    lcao = (1.0 + tbw) * s_ref[...]
    n2 = jnp.sum(lcao * lcao, axis=-1, keepdims=True)
    lcao = lcao * jax.lax.rsqrt(jnp.maximum(n2, 1e-24))
    lcao = jnp.dot(lcao, wb_ref[...], preferred_element_type=jnp.float32)
    nf1 = (jnp.dot(nfa_ref[...], wn1a_ref[...],
                   preferred_element_type=jnp.float32)
           + jnp.dot(nfb_ref[...], wn1b_ref[...],
                     preferred_element_type=jnp.float32)
           + bn1_ref[...])
    nf = _silu(jnp.dot(_silu(nf1), wn2_ref[...],
                       preferred_element_type=jnp.float32)
               + bn2_ref[...])
    msg_ref[...] = lcao * nf


def _post(agg_e, s, nf_a, nf_b, W_t1, b_t1, W_t2, b_t2, W_basis,
          W_n1, b_n1, W_n2, b_n2):
    return pl.pallas_call(
        _post_body,
        grid=(E // BE_P,),
        in_specs=[
            pl.BlockSpec((BE_P, VD), lambda i: (i, 0)),
            pl.BlockSpec((BE_P, VD), lambda i: (i, 0)),
            pl.BlockSpec((BE_P, VD), lambda i: (i, 0)),
            pl.BlockSpec((BE_P, VD), lambda i: (i, 0)),
            _full((VD, VD)), _full((1, VD)),
            _full((VD, VD)), _full((1, VD)),
            _full((VD, VD)),
            _full((VD, VD)), _full((VD, VD)), _full((1, VD)),
            _full((VD, VD)), _full((1, VD)),
        ],
        out_specs=pl.BlockSpec((BE_P, VD), lambda i: (i, 0)),
        out_shape=jax.ShapeDtypeStruct((E, VD), jnp.float32),
    )(agg_e, s, nf_a, nf_b, W_t1, b_t1.reshape(1, VD), W_t2,
      b_t2.reshape(1, VD), W_basis, W_n1[:VD], W_n1[VD:],
      b_n1.reshape(1, VD), W_n2, b_n2.reshape(1, VD))


def _final_body(x_ref, agg_ref, wo_ref, out_ref):
    agg = agg_ref[0] + agg_ref[1]
    out_ref[...] = x_ref[...] + jnp.dot(agg, wo_ref[...],
                                        preferred_element_type=jnp.float32)


def _final(x, agg_n2, W_out):
    return pl.pallas_call(
        _final_body,
        grid=(N // BN,),
        in_specs=[
            pl.BlockSpec((BN, HD), lambda i: (i, 0)),
            pl.BlockSpec((2, BN, VD), lambda i: (0, i, 0)),
            _full((VD, HD)),
        ],
        out_specs=pl.BlockSpec((BN, HD), lambda i: (i, 0)),
        out_shape=jax.ShapeDtypeStruct((N, HD), jnp.float32),
    )(x, agg_n2, W_out)


NW = 32
TPW = T // NW
BT = 200
NIT = TPW // BT


def _rsqrt_scalar(x):
    i = lax.bitcast_convert_type(x, jnp.int32)
    i = jnp.int32(0x5F3759DF) - (i >> 1)
    y = lax.bitcast_convert_type(i, jnp.float32)
    for _ in range(3):
        y = y * (1.5 - 0.5 * x * y * y)
    return y


def _tri_body(cw_hbm, shb_hbm, ekj_hbm, tk_hbm, xks_hbm, out_hbm,
              ekj_v, tk_v, rows_v, shb_v, xk_v, out_v, sem1, sem2):
    wid = lax.axis_index("s") * 2 + lax.axis_index("c")
    base = wid * TPW
    pltpu.sync_copy(ekj_hbm.at[pl.ds(base, TPW)], ekj_v)
    pltpu.sync_copy(tk_hbm.at[pl.ds(base, TPW)], tk_v)

    def body(it, carry):
        off = base + it * BT
        loc = it * BT
        pltpu.sync_copy(shb_hbm.at[pl.ds(off * NORB, BT * NORB)],
                        shb_v.at[pl.ds(0, BT * NORB)])
        cp1 = pltpu.async_copy(cw_hbm.at[ekj_v.at[pl.ds(loc, BT)]], rows_v,
                               sem1)
        cp2 = pltpu.async_copy(xks_hbm.at[tk_v.at[pl.ds(loc, BT)]], xk_v,
                               sem2)
        cp1.wait()
        cp2.wait()

        def per_t(t, tcarry):
            acc0 = jnp.zeros((16,), jnp.float32)
            acc1 = jnp.zeros((16,), jnp.float32)
            shrow = shb_v[pl.ds(t * NORB, 16)]
            for d in range(NORB):
                sh = shrow[d]
                acc0 = acc0 + sh * rows_v[t, pl.ds(d * VD, 16)]
                acc1 = acc1 + sh * rows_v[t, pl.ds(d * VD + 16, 16)]
            nsq = jnp.sum(acc0 * acc0 + acc1 * acc1, axis=0)
            rinv = _rsqrt_scalar(jnp.maximum(nsq, 1e-24))
            out_v[t, pl.ds(0, 16)] = acc0 * rinv * xk_v[t, pl.ds(0, 16)]
            out_v[t, pl.ds(16, 16)] = acc1 * rinv * xk_v[t, pl.ds(16, 16)]
            return tcarry

        lax.fori_loop(0, BT, per_t, 0, unroll=2)
        pltpu.sync_copy(out_v, out_hbm.at[pl.ds(off, BT)])
        return carry

    lax.fori_loop(0, NIT, body, 0)


def _tri_sc(cw, shb, edge_idx_kj, tri_idx_k, xks):
    mesh = plsc.VectorSubcoreMesh(core_axis_name="c", subcore_axis_name="s")
    f = pl.kernel(
        _tri_body,
        out_type=jax.ShapeDtypeStruct((T, VD), jnp.float32),
        mesh=mesh,
        compiler_params=pltpu.CompilerParams(needs_layout_passes=False,
                                             use_tc_tiling_on_sc=False),
        scratch_types=[
            pltpu.VMEM((TPW,), jnp.int32),
            pltpu.VMEM((TPW,), jnp.int32),
            pltpu.VMEM((BT, NORB * VD), jnp.float32),
            pltpu.VMEM((BT * NORB + 16,), jnp.float32),
            pltpu.VMEM((BT, VD), jnp.float32),
            pltpu.VMEM((BT, VD), jnp.float32),
            pltpu.SemaphoreType.DMA,
            pltpu.SemaphoreType.DMA,
        ],
    )
    return f(cw, shb, edge_idx_kj, tri_idx_k, xks)


ECH = 40000
NDUMP = 64
BV = 400
TPT = T // 16
ZROWS = (ECH + NDUMP) // 16
WROWS = ECH // 16


def _seg_e_body(tbw_hbm, eji_hbm, agg_hbm, idx_v, idx2_v, vals_v, zbuf_v,
                acc_sh, sem1):
    c = lax.axis_index("c")
    s = lax.axis_index("s")
    zero16 = jnp.zeros((16,), jnp.float32)

    def zrow(r, carry):
        zbuf_v[r, pl.ds(0, 16)] = zero16
        zbuf_v[r, pl.ds(16, 16)] = zero16
        return carry

    lax.fori_loop(0, BV, zrow, 0)

    for chunk_i in range(2):
        eb = (c * 2 + chunk_i) * ECH
        zb = s * ZROWS
        for j in range(ZROWS // BV):
            pltpu.sync_copy(zbuf_v, acc_sh.at[pl.ds(zb + j * BV, BV)])
        rem = ZROWS % BV
        pltpu.sync_copy(zbuf_v.at[pl.ds(0, rem)],
                        acc_sh.at[pl.ds(zb + (ZROWS // BV) * BV, rem)])
        plsc.subcore_barrier()

        tbase = s * TPT

        def rnd(r, carry):
            off = tbase + r * BV
            pltpu.sync_copy(eji_hbm.at[pl.ds(off, BV)], idx_v)
            pltpu.sync_copy(tbw_hbm.at[pl.ds(off, BV)], vals_v)

            def ix(j, jcarry):
                raw = idx_v[pl.ds(j * 16, 16)]
                v = raw - eb
                ok = (v >= 0) & (v < ECH)
                dump = ECH + (raw & (NDUMP - 1))
                idx2_v[pl.ds(j * 16, 16)] = jnp.where(ok, v, dump)
                return jcarry

            lax.fori_loop(0, BV // 16, ix, 0)
            pltpu.sync_copy(vals_v, acc_sh.at[idx2_v], add=True)
            return carry

        lax.fori_loop(0, TPT // BV, rnd, 0)
        plsc.subcore_barrier()

        wb = s * WROWS
        for j in range(WROWS // BV):
            pltpu.sync_copy(acc_sh.at[pl.ds(wb + j * BV, BV)], vals_v)
            pltpu.sync_copy(vals_v, agg_hbm.at[pl.ds(eb + wb + j * BV, BV)])
        remw = WROWS % BV
        pltpu.sync_copy(acc_sh.at[pl.ds(wb + (WROWS // BV) * BV, remw)],
                        vals_v.at[pl.ds(0, remw)])
        pltpu.sync_copy(vals_v.at[pl.ds(0, remw)],
                        agg_hbm.at[pl.ds(eb + wb + (WROWS // BV) * BV, remw)])
        plsc.subcore_barrier()


def _seg_e_sc(tbw_t, edge_idx_ji):
    mesh = plsc.VectorSubcoreMesh(core_axis_name="c", subcore_axis_name="s")
    f = pl.kernel(
        _seg_e_body,
        out_type=jax.ShapeDtypeStruct((E, VD), jnp.float32),
        mesh=mesh,
        compiler_params=pltpu.CompilerParams(needs_layout_passes=False,
                                             use_tc_tiling_on_sc=False),
        scratch_types=[
            pltpu.VMEM((BV,), jnp.int32),
            pltpu.VMEM((BV,), jnp.int32),
            pltpu.VMEM((BV, VD), jnp.float32),
            pltpu.VMEM((BV, VD), jnp.float32),
            pltpu.VMEM_SHARED((ECH + NDUMP, VD), jnp.float32),
            pltpu.SemaphoreType.DMA,
        ],
    )
    return f(tbw_t, edge_idx_ji)


EPW = E // NW
BG = 200


def _pair_body(x1_hbm, ii_hbm, jj_hbm, nfa_hbm, nfb_hbm,
               ii_v, jj_v, ra_v, rb_v, sem1, sem2):
    wid = lax.axis_index("s") * 2 + lax.axis_index("c")
    base = wid * EPW

    def rnd(r, carry):
        off = base + r * BG
        pltpu.sync_copy(ii_hbm.at[pl.ds(off, BG)], ii_v)
        pltpu.sync_copy(jj_hbm.at[pl.ds(off, BG)], jj_v)
        cp1 = pltpu.async_copy(x1_hbm.at[ii_v], ra_v, sem1)
        cp2 = pltpu.async_copy(x1_hbm.at[jj_v], rb_v, sem2)
        cp1.wait()
        cp2.wait()
        pltpu.sync_copy(ra_v, nfa_hbm.at[pl.ds(off, BG)])
        pltpu.sync_copy(rb_v, nfb_hbm.at[pl.ds(off, BG)])
        return carry

    lax.fori_loop(0, EPW // BG, rnd, 0)


def _pair_sc(x1, idx_i, idx_j):
    mesh = plsc.VectorSubcoreMesh(core_axis_name="c", subcore_axis_name="s")
    f = pl.kernel(
        _pair_body,
        out_type=[jax.ShapeDtypeStruct((E, VD), jnp.float32),
                  jax.ShapeDtypeStruct((E, VD), jnp.float32)],
        mesh=mesh,
        compiler_params=pltpu.CompilerParams(needs_layout_passes=False,
                                             use_tc_tiling_on_sc=False),
        scratch_types=[
            pltpu.VMEM((BG,), jnp.int32),
            pltpu.VMEM((BG,), jnp.int32),
            pltpu.VMEM((BG, VD), jnp.float32),
            pltpu.VMEM((BG, VD), jnp.float32),
            pltpu.SemaphoreType.DMA,
            pltpu.SemaphoreType.DMA,
        ],
    )
    return f(x1, idx_i, idx_j)


EPC = E // 2
EPT = EPC // 16
NZR = N // 16
BVN = 200


def _seg_n_body(msg_hbm, ii_hbm, agg_hbm, idx_v, vals_v, zbuf_v, acc_sh,
                sem1):
    c = lax.axis_index("c")
    s = lax.axis_index("s")
    zero16 = jnp.zeros((16,), jnp.float32)

    def zrow(r, carry):
        zbuf_v[r, pl.ds(0, 16)] = zero16
        zbuf_v[r, pl.ds(16, 16)] = zero16
        return carry

    lax.fori_loop(0, BVN, zrow, 0)
    zb = s * NZR
    for j in range(NZR // BVN):
        pltpu.sync_copy(zbuf_v, acc_sh.at[pl.ds(zb + j * BVN, BVN)])
    remz = NZR % BVN
    pltpu.sync_copy(zbuf_v.at[pl.ds(0, remz)],
                    acc_sh.at[pl.ds(zb + NZR - remz, remz)])
    plsc.subcore_barrier()

    tbase = c * EPC + s * EPT

    def rnd(r, carry):
        off = tbase + r * BVN
        pltpu.sync_copy(ii_hbm.at[pl.ds(off, BVN)], idx_v)
        pltpu.sync_copy(msg_hbm.at[pl.ds(off, BVN)], vals_v)
        pltpu.sync_copy(vals_v, acc_sh.at[idx_v], add=True)
        return carry

    lax.fori_loop(0, EPT // BVN, rnd, 0)
    plsc.subcore_barrier()

    for j in range(NZR // BVN):
        pltpu.sync_copy(acc_sh.at[pl.ds(zb + j * BVN, BVN)], vals_v)
        pltpu.sync_copy(vals_v, agg_hbm.at[c, pl.ds(zb + j * BVN, BVN)])
    pltpu.sync_copy(acc_sh.at[pl.ds(zb + NZR - remz, remz)],
                    vals_v.at[pl.ds(0, remz)])
    pltpu.sync_copy(vals_v.at[pl.ds(0, remz)],
                    agg_hbm.at[c, pl.ds(zb + NZR - remz, remz)])


def _seg_n_sc(msg, idx_i):
    mesh = plsc.VectorSubcoreMesh(core_axis_name="c", subcore_axis_name="s")
    f = pl.kernel(
        _seg_n_body,
        out_type=jax.ShapeDtypeStruct((2, N, VD), jnp.float32),
        mesh=mesh,
        compiler_params=pltpu.CompilerParams(needs_layout_passes=False,
                                             use_tc_tiling_on_sc=False),
        scratch_types=[
            pltpu.VMEM((BVN,), jnp.int32),
            pltpu.VMEM((BVN, VD), jnp.float32),
            pltpu.VMEM((BVN, VD), jnp.float32),
            pltpu.VMEM_SHARED((N, VD), jnp.float32),
            pltpu.SemaphoreType.DMA,
        ],
    )
    return f(msg, idx_i)


def kernel(x, cji, valence_mask, cutoff_w, rb, shb, idx_i, idx_j, tri_idx_k,
           edge_idx_kj, edge_idx_ji, W_node, b_node, W_c1, W_c2, W_t1, b_t1,
           W_t2, b_t2, W_basis, W_n1, b_n1, W_n2, b_n2, W_out):
    x1, xks = _prenode(x, W_node, b_node)
    cw, s = _cmlp(cji, rb, cutoff_w, W_c1, W_c2)

    tbw_t = _tri_sc(cw, shb.reshape(T * NORB), edge_idx_kj, tri_idx_k, xks)
    agg_e = _seg_e_sc(tbw_t, edge_idx_ji)

    nf_a, nf_b = _pair_sc(x1, idx_i, idx_j)

    msg = _post(agg_e, s, nf_a, nf_b, W_t1, b_t1, W_t2, b_t2, W_basis,
                W_n1, b_n1, W_n2, b_n2)

    agg_n2 = _seg_n_sc(msg, idx_i)
    return _final(x, agg_n2, W_out)


# speedup vs baseline: 1.4028x; 1.0132x over previous
import functools

import jax
import jax.numpy as jnp
from jax import lax
from jax.experimental import pallas as pl
from jax.experimental.pallas import tpu as pltpu
from jax.experimental.pallas import tpu_sc as plsc

N, E, T, NORB = 10000, 160000, 320000, 9
HD, CD, VD = 128, 16, 32

BN = 2000
BE_C = 2000
BE_P = 2000


def _silu(v):
    return v * jax.nn.sigmoid(v)


def _full(shape):
    return pl.BlockSpec(shape, lambda i: (0,) * len(shape))


def _prenode_body(x_ref, wn_ref, bn_ref, x1_ref, xks_ref):
    h = jnp.dot(x_ref[...], wn_ref[...], preferred_element_type=jnp.float32)
    h = h + bn_ref[...]
    x1_ref[...] = h[:, :VD]
    xks_ref[...] = jax.nn.sigmoid(h[:, VD:])


def _prenode(x, W_node, b_node):
    return pl.pallas_call(
        _prenode_body,
        grid=(N // BN,),
        in_specs=[
            pl.BlockSpec((BN, HD), lambda i: (i, 0)),
            _full((HD, 2 * VD)),
            _full((1, 2 * VD)),
        ],
        out_specs=[
            pl.BlockSpec((BN, VD), lambda i: (i, 0)),
            pl.BlockSpec((BN, VD), lambda i: (i, 0)),
        ],
        out_shape=[
            jax.ShapeDtypeStruct((N, VD), jnp.float32),
            jax.ShapeDtypeStruct((N, VD), jnp.float32),
        ],
    )(x, W_node, b_node.reshape(1, 2 * VD))


def _cmlp_body(cji_ref, rb_ref, cut_ref, w1_ref, w2_ref, r_ref, sm_ref,
               cw_ref, s_ref):
    z = _silu(jnp.dot(cji_ref[...], w1_ref[...],
                      preferred_element_type=jnp.float32))
    z = _silu(jnp.dot(z, w2_ref[...], preferred_element_type=jnp.float32))
    rbw = jnp.dot(rb_ref[...] * cut_ref[...], r_ref[...],
                  preferred_element_type=jnp.float32)
    cw = z * rbw
    cw_ref[...] = cw.astype(jnp.bfloat16)
    s_ref[...] = jnp.dot(cw, sm_ref[...], preferred_element_type=jnp.float32)


def _cmlp(cji, rb, cutoff_w, W_c1, W_c2):
    z16 = jnp.zeros((CD, VD), jnp.float32)
    z32 = jnp.zeros((VD, VD), jnp.float32)
    w1bd = jnp.concatenate(
        [jnp.concatenate([W_c1 if i == d else z16 for i in range(NORB)],
                         axis=1) for d in range(NORB)], axis=0)
    w2bd = jnp.concatenate(
        [jnp.concatenate([W_c2 if i == d else z32 for i in range(NORB)],
                         axis=1) for d in range(NORB)], axis=0)
    rmap = jnp.repeat(jnp.eye(NORB, dtype=jnp.float32), VD, axis=1)
    smap = jnp.tile(jnp.eye(VD, dtype=jnp.float32), (NORB, 1))
    return pl.pallas_call(
        _cmlp_body,
        grid=(E // BE_C,),
        in_specs=[
            pl.BlockSpec((BE_C, NORB * CD), lambda i: (i, 0)),
            pl.BlockSpec((BE_C, NORB), lambda i: (i, 0)),
            pl.BlockSpec((BE_C, 1), lambda i: (i, 0)),
            _full((NORB * CD, NORB * VD)),
            _full((NORB * VD, NORB * VD)),
            _full((NORB, NORB * VD)),
            _full((NORB * VD, VD)),
        ],
        out_specs=[
            pl.BlockSpec((BE_C, NORB * VD), lambda i: (i, 0)),
            pl.BlockSpec((BE_C, VD), lambda i: (i, 0)),
        ],
        out_shape=[
            jax.ShapeDtypeStruct((E, NORB * VD), jnp.bfloat16),
            jax.ShapeDtypeStruct((E, VD), jnp.float32),
        ],
    )(cji.reshape(E, NORB * CD), rb, cutoff_w.reshape(E, 1),
      w1bd, w2bd, rmap, smap)


def _post_body(agg_ref, s_ref, nfa_ref, nfb_ref, wt1_ref, bt1_ref, wt2_ref,
               bt2_ref, wb_ref, wn1a_ref, wn1b_ref, bn1_ref, wn2_ref,
               bn2_ref, msg_ref):
    tbw = _silu(jnp.dot(_silu(jnp.dot(agg_ref[...], wt1_ref[...],
                                      preferred_element_type=jnp.float32)
                              + bt1_ref[...]),
                        wt2_ref[...], preferred_element_type=jnp.float32)
                + bt2_ref[...])
    lcao = (1.0 + tbw) * s_ref[...]
    n2 = jnp.sum(lcao * lcao, axis=-1, keepdims=True)
    lcao = lcao * jax.lax.rsqrt(jnp.maximum(n2, 1e-24))
    lcao = jnp.dot(lcao, wb_ref[...], preferred_element_type=jnp.float32)
    nf1 = (jnp.dot(nfa_ref[...], wn1a_ref[...],
                   preferred_element_type=jnp.float32)
           + jnp.dot(nfb_ref[...], wn1b_ref[...],
                     preferred_element_type=jnp.float32)
           + bn1_ref[...])
    nf = _silu(jnp.dot(_silu(nf1), wn2_ref[...],
                       preferred_element_type=jnp.float32)
               + bn2_ref[...])
    msg_ref[...] = lcao * nf


def _post(agg_e, s, nf_a, nf_b, W_t1, b_t1, W_t2, b_t2, W_basis,
          W_n1, b_n1, W_n2, b_n2):
    return pl.pallas_call(
        _post_body,
        grid=(E // BE_P,),
        in_specs=[
            pl.BlockSpec((BE_P, VD), lambda i: (i, 0)),
            pl.BlockSpec((BE_P, VD), lambda i: (i, 0)),
            pl.BlockSpec((BE_P, VD), lambda i: (i, 0)),
            pl.BlockSpec((BE_P, VD), lambda i: (i, 0)),
            _full((VD, VD)), _full((1, VD)),
            _full((VD, VD)), _full((1, VD)),
            _full((VD, VD)),
            _full((VD, VD)), _full((VD, VD)), _full((1, VD)),
            _full((VD, VD)), _full((1, VD)),
        ],
        out_specs=pl.BlockSpec((BE_P, VD), lambda i: (i, 0)),
        out_shape=jax.ShapeDtypeStruct((E, VD), jnp.float32),
    )(agg_e, s, nf_a, nf_b, W_t1, b_t1.reshape(1, VD), W_t2,
      b_t2.reshape(1, VD), W_basis, W_n1[:VD], W_n1[VD:],
      b_n1.reshape(1, VD), W_n2, b_n2.reshape(1, VD))


def _final_body(x_ref, agg_ref, wo_ref, out_ref):
    agg = agg_ref[0] + agg_ref[1]
    out_ref[...] = x_ref[...] + jnp.dot(agg, wo_ref[...],
                                        preferred_element_type=jnp.float32)


def _final(x, agg_n2, W_out):
    return pl.pallas_call(
        _final_body,
        grid=(N // BN,),
        in_specs=[
            pl.BlockSpec((BN, HD), lambda i: (i, 0)),
            pl.BlockSpec((2, BN, VD), lambda i: (0, i, 0)),
            _full((VD, HD)),
        ],
        out_specs=pl.BlockSpec((BN, HD), lambda i: (i, 0)),
        out_shape=jax.ShapeDtypeStruct((N, HD), jnp.float32),
    )(x, agg_n2, W_out)


NW = 32
TPW = T // NW
BT = 200
NIT = TPW // BT


def _rsqrt_scalar(x):
    i = lax.bitcast_convert_type(x, jnp.int32)
    i = jnp.int32(0x5F3759DF) - (i >> 1)
    y = lax.bitcast_convert_type(i, jnp.float32)
    for _ in range(3):
        y = y * (1.5 - 0.5 * x * y * y)
    return y


def _tri_body(cw_hbm, shb_hbm, ekj_hbm, tk_hbm, xks_hbm, out_hbm,
              ekj_v, tk_v, rows_v, shb_v, xk_v, out_v, sem1, sem2):
    wid = lax.axis_index("s") * 2 + lax.axis_index("c")
    base = wid * TPW
    pltpu.sync_copy(ekj_hbm.at[pl.ds(base, TPW)], ekj_v)
    pltpu.sync_copy(tk_hbm.at[pl.ds(base, TPW)], tk_v)

    def body(it, carry):
        off = base + it * BT
        loc = it * BT
        pltpu.sync_copy(shb_hbm.at[pl.ds(off * NORB, BT * NORB)],
                        shb_v.at[pl.ds(0, BT * NORB)])
        cp1 = pltpu.async_copy(cw_hbm.at[ekj_v.at[pl.ds(loc, BT)]], rows_v,
                               sem1)
        cp2 = pltpu.async_copy(xks_hbm.at[tk_v.at[pl.ds(loc, BT)]], xk_v,
                               sem2)
        cp1.wait()
        cp2.wait()

        def per_t(t, tcarry):
            acc0 = jnp.zeros((16,), jnp.float32)
            acc1 = jnp.zeros((16,), jnp.float32)
            shrow = shb_v[pl.ds(t * NORB, 16)]
            for d in range(NORB):
                sh = shrow[d]
                v = rows_v[t, pl.ds(d * VD, VD)]
                a, b = plsc.unpack(v, format=plsc.PackFormat.INTERLEAVED)
                acc0 = acc0 + sh * a
                acc1 = acc1 + sh * b
            nsq = jnp.sum(acc0 * acc0 + acc1 * acc1, axis=0)
            rinv = _rsqrt_scalar(jnp.maximum(nsq, 1e-24))
            out_v[t, pl.ds(0, 16)] = acc0 * rinv * xk_v[t, pl.ds(0, 16)]
            out_v[t, pl.ds(16, 16)] = acc1 * rinv * xk_v[t, pl.ds(16, 16)]
            return tcarry

        lax.fori_loop(0, BT, per_t, 0, unroll=2)
        pltpu.sync_copy(out_v, out_hbm.at[pl.ds(off, BT)])
        return carry

    lax.fori_loop(0, NIT, body, 0)


def _tri_sc(cw, shb, edge_idx_kj, tri_idx_k, xks):
    mesh = plsc.VectorSubcoreMesh(core_axis_name="c", subcore_axis_name="s")
    f = pl.kernel(
        _tri_body,
        out_type=jax.ShapeDtypeStruct((T, VD), jnp.float32),
        mesh=mesh,
        compiler_params=pltpu.CompilerParams(needs_layout_passes=False,
                                             use_tc_tiling_on_sc=False),
        scratch_types=[
            pltpu.VMEM((TPW,), jnp.int32),
            pltpu.VMEM((TPW,), jnp.int32),
            pltpu.VMEM((BT, NORB * VD), jnp.bfloat16),
            pltpu.VMEM((BT * NORB + 16,), jnp.float32),
            pltpu.VMEM((BT, VD), jnp.float32),
            pltpu.VMEM((BT, VD), jnp.float32),
            pltpu.SemaphoreType.DMA,
            pltpu.SemaphoreType.DMA,
        ],
    )
    return f(cw, shb, edge_idx_kj, tri_idx_k, xks)


ECH = 40000
NDUMP = 64
BV = 400
TPT = T // 16
ZROWS = (ECH + NDUMP) // 16
WROWS = ECH // 16


def _seg_e_body(tbw_hbm, eji_hbm, agg_hbm, idx_v, idx2_v, vals_v, zbuf_v,
                acc_sh, sem1):
    c = lax.axis_index("c")
    s = lax.axis_index("s")
    zero16 = jnp.zeros((16,), jnp.float32)

    def zrow(r, carry):
        zbuf_v[r, pl.ds(0, 16)] = zero16
        zbuf_v[r, pl.ds(16, 16)] = zero16
        return carry

    lax.fori_loop(0, BV, zrow, 0)

    for chunk_i in range(2):
        eb = (c * 2 + chunk_i) * ECH
        zb = s * ZROWS
        for j in range(ZROWS // BV):
            pltpu.sync_copy(zbuf_v, acc_sh.at[pl.ds(zb + j * BV, BV)])
        rem = ZROWS % BV
        pltpu.sync_copy(zbuf_v.at[pl.ds(0, rem)],
                        acc_sh.at[pl.ds(zb + (ZROWS // BV) * BV, rem)])
        plsc.subcore_barrier()

        tbase = s * TPT

        def rnd(r, carry):
            off = tbase + r * BV
            pltpu.sync_copy(eji_hbm.at[pl.ds(off, BV)], idx_v)
            pltpu.sync_copy(tbw_hbm.at[pl.ds(off, BV)], vals_v)

            def ix(j, jcarry):
                raw = idx_v[pl.ds(j * 16, 16)]
                v = raw - eb
                ok = (v >= 0) & (v < ECH)
                dump = ECH + (raw & (NDUMP - 1))
                idx2_v[pl.ds(j * 16, 16)] = jnp.where(ok, v, dump)
                return jcarry

            lax.fori_loop(0, BV // 16, ix, 0)
            pltpu.sync_copy(vals_v, acc_sh.at[idx2_v], add=True)
            return carry

        lax.fori_loop(0, TPT // BV, rnd, 0)
        plsc.subcore_barrier()

        wb = s * WROWS
        for j in range(WROWS // BV):
            pltpu.sync_copy(acc_sh.at[pl.ds(wb + j * BV, BV)], vals_v)
            pltpu.sync_copy(vals_v, agg_hbm.at[pl.ds(eb + wb + j * BV, BV)])
        remw = WROWS % BV
        pltpu.sync_copy(acc_sh.at[pl.ds(wb + (WROWS // BV) * BV, remw)],
                        vals_v.at[pl.ds(0, remw)])
        pltpu.sync_copy(vals_v.at[pl.ds(0, remw)],
                        agg_hbm.at[pl.ds(eb + wb + (WROWS // BV) * BV, remw)])
        plsc.subcore_barrier()


def _seg_e_sc(tbw_t, edge_idx_ji):
    mesh = plsc.VectorSubcoreMesh(core_axis_name="c", subcore_axis_name="s")
    f = pl.kernel(
        _seg_e_body,
        out_type=jax.ShapeDtypeStruct((E, VD), jnp.float32),
        mesh=mesh,
        compiler_params=pltpu.CompilerParams(needs_layout_passes=False,
                                             use_tc_tiling_on_sc=False),
        scratch_types=[
            pltpu.VMEM((BV,), jnp.int32),
            pltpu.VMEM((BV,), jnp.int32),
            pltpu.VMEM((BV, VD), jnp.float32),
            pltpu.VMEM((BV, VD), jnp.float32),
            pltpu.VMEM_SHARED((ECH + NDUMP, VD), jnp.float32),
            pltpu.SemaphoreType.DMA,
        ],
    )
    return f(tbw_t, edge_idx_ji)


EPW = E // NW
BG = 200


def _pair_body(x1_hbm, ii_hbm, jj_hbm, nfa_hbm, nfb_hbm,
               ii_v, jj_v, ra_v, rb_v, sem1, sem2):
    wid = lax.axis_index("s") * 2 + lax.axis_index("c")
    base = wid * EPW

    def rnd(r, carry):
        off = base + r * BG
        pltpu.sync_copy(ii_hbm.at[pl.ds(off, BG)], ii_v)
        pltpu.sync_copy(jj_hbm.at[pl.ds(off, BG)], jj_v)
        cp1 = pltpu.async_copy(x1_hbm.at[ii_v], ra_v, sem1)
        cp2 = pltpu.async_copy(x1_hbm.at[jj_v], rb_v, sem2)
        cp1.wait()
        cp2.wait()
        pltpu.sync_copy(ra_v, nfa_hbm.at[pl.ds(off, BG)])
        pltpu.sync_copy(rb_v, nfb_hbm.at[pl.ds(off, BG)])
        return carry

    lax.fori_loop(0, EPW // BG, rnd, 0)


def _pair_sc(x1, idx_i, idx_j):
    mesh = plsc.VectorSubcoreMesh(core_axis_name="c", subcore_axis_name="s")
    f = pl.kernel(
        _pair_body,
        out_type=[jax.ShapeDtypeStruct((E, VD), jnp.float32),
                  jax.ShapeDtypeStruct((E, VD), jnp.float32)],
        mesh=mesh,
        compiler_params=pltpu.CompilerParams(needs_layout_passes=False,
                                             use_tc_tiling_on_sc=False),
        scratch_types=[
            pltpu.VMEM((BG,), jnp.int32),
            pltpu.VMEM((BG,), jnp.int32),
            pltpu.VMEM((BG, VD), jnp.float32),
            pltpu.VMEM((BG, VD), jnp.float32),
            pltpu.SemaphoreType.DMA,
            pltpu.SemaphoreType.DMA,
        ],
    )
    return f(x1, idx_i, idx_j)


EPC = E // 2
EPT = EPC // 16
NZR = N // 16
BVN = 200


def _seg_n_body(msg_hbm, ii_hbm, agg_hbm, idx_v, vals_v, zbuf_v, acc_sh,
                sem1):
    c = lax.axis_index("c")
    s = lax.axis_index("s")
    zero16 = jnp.zeros((16,), jnp.float32)

    def zrow(r, carry):
        zbuf_v[r, pl.ds(0, 16)] = zero16
        zbuf_v[r, pl.ds(16, 16)] = zero16
        return carry

    lax.fori_loop(0, BVN, zrow, 0)
    zb = s * NZR
    for j in range(NZR // BVN):
        pltpu.sync_copy(zbuf_v, acc_sh.at[pl.ds(zb + j * BVN, BVN)])
    remz = NZR % BVN
    pltpu.sync_copy(zbuf_v.at[pl.ds(0, remz)],
                    acc_sh.at[pl.ds(zb + NZR - remz, remz)])
    plsc.subcore_barrier()

    tbase = c * EPC + s * EPT

    def rnd(r, carry):
        off = tbase + r * BVN
        pltpu.sync_copy(ii_hbm.at[pl.ds(off, BVN)], idx_v)
        pltpu.sync_copy(msg_hbm.at[pl.ds(off, BVN)], vals_v)
        pltpu.sync_copy(vals_v, acc_sh.at[idx_v], add=True)
        return carry

    lax.fori_loop(0, EPT // BVN, rnd, 0)
    plsc.subcore_barrier()

    for j in range(NZR // BVN):
        pltpu.sync_copy(acc_sh.at[pl.ds(zb + j * BVN, BVN)], vals_v)
        pltpu.sync_copy(vals_v, agg_hbm.at[c, pl.ds(zb + j * BVN, BVN)])
    pltpu.sync_copy(acc_sh.at[pl.ds(zb + NZR - remz, remz)],
                    vals_v.at[pl.ds(0, remz)])
    pltpu.sync_copy(vals_v.at[pl.ds(0, remz)],
                    agg_hbm.at[c, pl.ds(zb + NZR - remz, remz)])


def _seg_n_sc(msg, idx_i):
    mesh = plsc.VectorSubcoreMesh(core_axis_name="c", subcore_axis_name="s")
    f = pl.kernel(
        _seg_n_body,
        out_type=jax.ShapeDtypeStruct((2, N, VD), jnp.float32),
        mesh=mesh,
        compiler_params=pltpu.CompilerParams(needs_layout_passes=False,
                                             use_tc_tiling_on_sc=False),
        scratch_types=[
            pltpu.VMEM((BVN,), jnp.int32),
            pltpu.VMEM((BVN, VD), jnp.float32),
            pltpu.VMEM((BVN, VD), jnp.float32),
            pltpu.VMEM_SHARED((N, VD), jnp.float32),
            pltpu.SemaphoreType.DMA,
        ],
    )
    return f(msg, idx_i)


def kernel(x, cji, valence_mask, cutoff_w, rb, shb, idx_i, idx_j, tri_idx_k,
           edge_idx_kj, edge_idx_ji, W_node, b_node, W_c1, W_c2, W_t1, b_t1,
           W_t2, b_t2, W_basis, W_n1, b_n1, W_n2, b_n2, W_out):
    order = jnp.arange(VD, dtype=jnp.int32).reshape(VD // 2, 2).T.reshape(VD)
    wn_p = jnp.concatenate([W_node[:, :VD], W_node[:, VD:][:, order]], axis=1)
    bn_p = jnp.concatenate([b_node[:VD], b_node[VD:][order]])
    x1, xks = _prenode(x, wn_p, bn_p)
    cw, s = _cmlp(cji, rb, cutoff_w, W_c1, W_c2)

    tbw_t = _tri_sc(cw, shb.reshape(T * NORB), edge_idx_kj, tri_idx_k, xks)
    agg_e = _seg_e_sc(tbw_t, edge_idx_ji)

    nf_a, nf_b = _pair_sc(x1, idx_i, idx_j)

    msg = _post(agg_e, s, nf_a, nf_b, W_t1[order], b_t1, W_t2, b_t2,
                W_basis, W_n1, b_n1, W_n2, b_n2)

    agg_n2 = _seg_n_sc(msg, idx_i)
    return _final(x, agg_n2, W_out)


# speedup vs baseline: 1.4150x; 1.0088x over previous
import functools

import jax
import jax.numpy as jnp
from jax import lax
from jax.experimental import pallas as pl
from jax.experimental.pallas import tpu as pltpu
from jax.experimental.pallas import tpu_sc as plsc

N, E, T, NORB = 10000, 160000, 320000, 9
HD, CD, VD = 128, 16, 32

BN = 2000
BE_C = 2000
BE_P = 2000


def _silu(v):
    return v * jax.nn.sigmoid(v)


def _full(shape):
    return pl.BlockSpec(shape, lambda i: (0,) * len(shape))


def _prenode_body(x_ref, wn_ref, bn_ref, x1_ref, xks_ref):
    h = jnp.dot(x_ref[...], wn_ref[...], preferred_element_type=jnp.float32)
    h = h + bn_ref[...]
    x1_ref[...] = h[:, :VD]
    xks_ref[...] = jax.nn.sigmoid(h[:, VD:])


def _prenode(x, W_node, b_node):
    return pl.pallas_call(
        _prenode_body,
        grid=(N // BN,),
        in_specs=[
            pl.BlockSpec((BN, HD), lambda i: (i, 0)),
            _full((HD, 2 * VD)),
            _full((1, 2 * VD)),
        ],
        out_specs=[
            pl.BlockSpec((BN, VD), lambda i: (i, 0)),
            pl.BlockSpec((BN, VD), lambda i: (i, 0)),
        ],
        out_shape=[
            jax.ShapeDtypeStruct((N, VD), jnp.float32),
            jax.ShapeDtypeStruct((N, VD), jnp.float32),
        ],
    )(x, W_node, b_node.reshape(1, 2 * VD))


def _cmlp_body(cji_ref, rb_ref, cut_ref, w1_ref, w2_ref, r_ref, sm_ref,
               cw_ref, s_ref):
    z = _silu(jnp.dot(cji_ref[...], w1_ref[...],
                      preferred_element_type=jnp.float32))
    z = _silu(jnp.dot(z, w2_ref[...], preferred_element_type=jnp.float32))
    rbw = jnp.dot(rb_ref[...] * cut_ref[...], r_ref[...],
                  preferred_element_type=jnp.float32)
    cw = z * rbw
    cw_ref[...] = cw.astype(jnp.bfloat16)
    s_ref[...] = jnp.dot(cw, sm_ref[...], preferred_element_type=jnp.float32)


def _cmlp(cji, rb, cutoff_w, W_c1, W_c2):
    z16 = jnp.zeros((CD, VD), jnp.float32)
    z32 = jnp.zeros((VD, VD), jnp.float32)
    w1bd = jnp.concatenate(
        [jnp.concatenate([W_c1 if i == d else z16 for i in range(NORB)],
                         axis=1) for d in range(NORB)], axis=0)
    w2bd = jnp.concatenate(
        [jnp.concatenate([W_c2 if i == d else z32 for i in range(NORB)],
                         axis=1) for d in range(NORB)], axis=0)
    rmap = jnp.repeat(jnp.eye(NORB, dtype=jnp.float32), VD, axis=1)
    smap = jnp.tile(jnp.eye(VD, dtype=jnp.float32), (NORB, 1))
    return pl.pallas_call(
        _cmlp_body,
        grid=(E // BE_C,),
        in_specs=[
            pl.BlockSpec((BE_C, NORB * CD), lambda i: (i, 0)),
            pl.BlockSpec((BE_C, NORB), lambda i: (i, 0)),
            pl.BlockSpec((BE_C, 1), lambda i: (i, 0)),
            _full((NORB * CD, NORB * VD)),
            _full((NORB * VD, NORB * VD)),
            _full((NORB, NORB * VD)),
            _full((NORB * VD, VD)),
        ],
        out_specs=[
            pl.BlockSpec((BE_C, NORB * VD), lambda i: (i, 0)),
            pl.BlockSpec((BE_C, VD), lambda i: (i, 0)),
        ],
        out_shape=[
            jax.ShapeDtypeStruct((E, NORB * VD), jnp.bfloat16),
            jax.ShapeDtypeStruct((E, VD), jnp.float32),
        ],
    )(cji.reshape(E, NORB * CD), rb, cutoff_w.reshape(E, 1),
      w1bd, w2bd, rmap, smap)


def _post_body(agg_ref, s_ref, nfa_ref, nfb_ref, wt1_ref, bt1_ref, wt2_ref,
               bt2_ref, wb_ref, wn1a_ref, wn1b_ref, bn1_ref, wn2_ref,
               bn2_ref, msg_ref):
    tbw = _silu(jnp.dot(_silu(jnp.dot(agg_ref[...], wt1_ref[...],
                                      preferred_element_type=jnp.float32)
                              + bt1_ref[...]),
                        wt2_ref[...], preferred_element_type=jnp.float32)
                + bt2_ref[...])
    lcao = (1.0 + tbw) * s_ref[...]
    n2 = jnp.sum(lcao * lcao, axis=-1, keepdims=True)
    lcao = lcao * jax.lax.rsqrt(jnp.maximum(n2, 1e-24))
    lcao = jnp.dot(lcao, wb_ref[...], preferred_element_type=jnp.float32)
    nf1 = (jnp.dot(nfa_ref[...], wn1a_ref[...],
                   preferred_element_type=jnp.float32)
           + jnp.dot(nfb_ref[...], wn1b_ref[...],
                     preferred_element_type=jnp.float32)
           + bn1_ref[...])
    nf = _silu(jnp.dot(_silu(nf1), wn2_ref[...],
                       preferred_element_type=jnp.float32)
               + bn2_ref[...])
    msg_ref[...] = lcao * nf


def _post(agg_e, s, nf_a, nf_b, W_t1, b_t1, W_t2, b_t2, W_basis,
          W_n1, b_n1, W_n2, b_n2):
    return pl.pallas_call(
        _post_body,
        grid=(E // BE_P,),
        in_specs=[
            pl.BlockSpec((BE_P, VD), lambda i: (i, 0)),
            pl.BlockSpec((BE_P, VD), lambda i: (i, 0)),
            pl.BlockSpec((BE_P, VD), lambda i: (i, 0)),
            pl.BlockSpec((BE_P, VD), lambda i: (i, 0)),
            _full((VD, VD)), _full((1, VD)),
            _full((VD, VD)), _full((1, VD)),
            _full((VD, VD)),
            _full((VD, VD)), _full((VD, VD)), _full((1, VD)),
            _full((VD, VD)), _full((1, VD)),
        ],
        out_specs=pl.BlockSpec((BE_P, VD), lambda i: (i, 0)),
        out_shape=jax.ShapeDtypeStruct((E, VD), jnp.float32),
    )(agg_e, s, nf_a, nf_b, W_t1, b_t1.reshape(1, VD), W_t2,
      b_t2.reshape(1, VD), W_basis, W_n1[:VD], W_n1[VD:],
      b_n1.reshape(1, VD), W_n2, b_n2.reshape(1, VD))


def _final_body(x_ref, agg_ref, wo_ref, out_ref):
    agg = agg_ref[0] + agg_ref[1]
    out_ref[...] = x_ref[...] + jnp.dot(agg, wo_ref[...],
                                        preferred_element_type=jnp.float32)


def _final(x, agg_n2, W_out):
    return pl.pallas_call(
        _final_body,
        grid=(N // BN,),
        in_specs=[
            pl.BlockSpec((BN, HD), lambda i: (i, 0)),
            pl.BlockSpec((2, BN, VD), lambda i: (0, i, 0)),
            _full((VD, HD)),
        ],
        out_specs=pl.BlockSpec((BN, HD), lambda i: (i, 0)),
        out_shape=jax.ShapeDtypeStruct((N, HD), jnp.float32),
    )(x, agg_n2, W_out)


NW = 32
TPW = T // NW
BT = 400
NIT = TPW // BT


def _rsqrt_scalar(x):
    i = lax.bitcast_convert_type(x, jnp.int32)
    i = jnp.int32(0x5F3759DF) - (i >> 1)
    y = lax.bitcast_convert_type(i, jnp.float32)
    for _ in range(3):
        y = y * (1.5 - 0.5 * x * y * y)
    return y


def _tri_body(cw_hbm, shb_hbm, ekj_hbm, tk_hbm, xks_hbm, out_hbm,
              ekj_v, tk_v, rows_v, shb_v, xk_v, out_v, sem1, sem2):
    wid = lax.axis_index("s") * 2 + lax.axis_index("c")
    base = wid * TPW
    pltpu.sync_copy(ekj_hbm.at[pl.ds(base, TPW)], ekj_v)
    pltpu.sync_copy(tk_hbm.at[pl.ds(base, TPW)], tk_v)

    def body(it, carry):
        off = base + it * BT
        loc = it * BT
        pltpu.sync_copy(shb_hbm.at[pl.ds(off * NORB, BT * NORB)],
                        shb_v.at[pl.ds(0, BT * NORB)])
        cp1 = pltpu.async_copy(cw_hbm.at[ekj_v.at[pl.ds(loc, BT)]], rows_v,
                               sem1)
        cp2 = pltpu.async_copy(xks_hbm.at[tk_v.at[pl.ds(loc, BT)]], xk_v,
                               sem2)
        cp1.wait()
        cp2.wait()

        def per_t(t, tcarry):
            acc0 = jnp.zeros((16,), jnp.float32)
            acc1 = jnp.zeros((16,), jnp.float32)
            shrow = shb_v[pl.ds(t * NORB, 16)]
            for d in range(NORB):
                sh = shrow[d]
                v = rows_v[t, pl.ds(d * VD, VD)]
                a, b = plsc.unpack(v, format=plsc.PackFormat.INTERLEAVED)
                acc0 = acc0 + sh * a
                acc1 = acc1 + sh * b
            nsq = jnp.sum(acc0 * acc0 + acc1 * acc1, axis=0)
            rinv = _rsqrt_scalar(jnp.maximum(nsq, 1e-24))
            out_v[t, pl.ds(0, 16)] = acc0 * rinv * xk_v[t, pl.ds(0, 16)]
            out_v[t, pl.ds(16, 16)] = acc1 * rinv * xk_v[t, pl.ds(16, 16)]
            return tcarry

        lax.fori_loop(0, BT, per_t, 0, unroll=2)
        pltpu.sync_copy(out_v, out_hbm.at[pl.ds(off, BT)])
        return carry

    lax.fori_loop(0, NIT, body, 0)


def _tri_sc(cw, shb, edge_idx_kj, tri_idx_k, xks):
    mesh = plsc.VectorSubcoreMesh(core_axis_name="c", subcore_axis_name="s")
    f = pl.kernel(
        _tri_body,
        out_type=jax.ShapeDtypeStruct((T, VD), jnp.float32),
        mesh=mesh,
        compiler_params=pltpu.CompilerParams(needs_layout_passes=False,
                                             use_tc_tiling_on_sc=False),
        scratch_types=[
            pltpu.VMEM((TPW,), jnp.int32),
            pltpu.VMEM((TPW,), jnp.int32),
            pltpu.VMEM((BT, NORB * VD), jnp.bfloat16),
            pltpu.VMEM((BT * NORB + 16,), jnp.float32),
            pltpu.VMEM((BT, VD), jnp.float32),
            pltpu.VMEM((BT, VD), jnp.float32),
            pltpu.SemaphoreType.DMA,
            pltpu.SemaphoreType.DMA,
        ],
    )
    return f(cw, shb, edge_idx_kj, tri_idx_k, xks)


ECH = 40000
NDUMP = 64
BV = 400
TPT = T // 16
ZROWS = (ECH + NDUMP) // 16
WROWS = ECH // 16


def _seg_e_body(tbw_hbm, eji_hbm, agg_hbm, idx_v, idx2_v, vals_v, zbuf_v,
                acc_sh, sem1):
    c = lax.axis_index("c")
    s = lax.axis_index("s")
    zero16 = jnp.zeros((16,), jnp.float32)

    def zrow(r, carry):
        zbuf_v[r, pl.ds(0, 16)] = zero16
        zbuf_v[r, pl.ds(16, 16)] = zero16
        return carry

    lax.fori_loop(0, BV, zrow, 0)

    for chunk_i in range(2):
        eb = (c * 2 + chunk_i) * ECH
        zb = s * ZROWS
        for j in range(ZROWS // BV):
            pltpu.sync_copy(zbuf_v, acc_sh.at[pl.ds(zb + j * BV, BV)])
        rem = ZROWS % BV
        pltpu.sync_copy(zbuf_v.at[pl.ds(0, rem)],
                        acc_sh.at[pl.ds(zb + (ZROWS // BV) * BV, rem)])
        plsc.subcore_barrier()

        tbase = s * TPT

        def rnd(r, carry):
            off = tbase + r * BV
            pltpu.sync_copy(eji_hbm.at[pl.ds(off, BV)], idx_v)
            pltpu.sync_copy(tbw_hbm.at[pl.ds(off, BV)], vals_v)

            def ix(j, jcarry):
                raw = idx_v[pl.ds(j * 16, 16)]
                v = raw - eb
                ok = (v >= 0) & (v < ECH)
                dump = ECH + (raw & (NDUMP - 1))
                idx2_v[pl.ds(j * 16, 16)] = jnp.where(ok, v, dump)
                return jcarry

            lax.fori_loop(0, BV // 16, ix, 0)
            pltpu.sync_copy(vals_v, acc_sh.at[idx2_v], add=True)
            return carry

        lax.fori_loop(0, TPT // BV, rnd, 0)
        plsc.subcore_barrier()

        wb = s * WROWS
        for j in range(WROWS // BV):
            pltpu.sync_copy(acc_sh.at[pl.ds(wb + j * BV, BV)], vals_v)
            pltpu.sync_copy(vals_v, agg_hbm.at[pl.ds(eb + wb + j * BV, BV)])
        remw = WROWS % BV
        pltpu.sync_copy(acc_sh.at[pl.ds(wb + (WROWS // BV) * BV, remw)],
                        vals_v.at[pl.ds(0, remw)])
        pltpu.sync_copy(vals_v.at[pl.ds(0, remw)],
                        agg_hbm.at[pl.ds(eb + wb + (WROWS // BV) * BV, remw)])
        plsc.subcore_barrier()


def _seg_e_sc(tbw_t, edge_idx_ji):
    mesh = plsc.VectorSubcoreMesh(core_axis_name="c", subcore_axis_name="s")
    f = pl.kernel(
        _seg_e_body,
        out_type=jax.ShapeDtypeStruct((E, VD), jnp.float32),
        mesh=mesh,
        compiler_params=pltpu.CompilerParams(needs_layout_passes=False,
                                             use_tc_tiling_on_sc=False),
        scratch_types=[
            pltpu.VMEM((BV,), jnp.int32),
            pltpu.VMEM((BV,), jnp.int32),
            pltpu.VMEM((BV, VD), jnp.float32),
            pltpu.VMEM((BV, VD), jnp.float32),
            pltpu.VMEM_SHARED((ECH + NDUMP, VD), jnp.float32),
            pltpu.SemaphoreType.DMA,
        ],
    )
    return f(tbw_t, edge_idx_ji)


EPW = E // NW
BG = 200


def _pair_body(x1_hbm, ii_hbm, jj_hbm, nfa_hbm, nfb_hbm,
               ii_v, jj_v, ra_v, rb_v, sem1, sem2):
    wid = lax.axis_index("s") * 2 + lax.axis_index("c")
    base = wid * EPW

    def rnd(r, carry):
        off = base + r * BG
        pltpu.sync_copy(ii_hbm.at[pl.ds(off, BG)], ii_v)
        pltpu.sync_copy(jj_hbm.at[pl.ds(off, BG)], jj_v)
        cp1 = pltpu.async_copy(x1_hbm.at[ii_v], ra_v, sem1)
        cp2 = pltpu.async_copy(x1_hbm.at[jj_v], rb_v, sem2)
        cp1.wait()
        cp2.wait()
        pltpu.sync_copy(ra_v, nfa_hbm.at[pl.ds(off, BG)])
        pltpu.sync_copy(rb_v, nfb_hbm.at[pl.ds(off, BG)])
        return carry

    lax.fori_loop(0, EPW // BG, rnd, 0)


def _pair_sc(x1, idx_i, idx_j):
    mesh = plsc.VectorSubcoreMesh(core_axis_name="c", subcore_axis_name="s")
    f = pl.kernel(
        _pair_body,
        out_type=[jax.ShapeDtypeStruct((E, VD), jnp.float32),
                  jax.ShapeDtypeStruct((E, VD), jnp.float32)],
        mesh=mesh,
        compiler_params=pltpu.CompilerParams(needs_layout_passes=False,
                                             use_tc_tiling_on_sc=False),
        scratch_types=[
            pltpu.VMEM((BG,), jnp.int32),
            pltpu.VMEM((BG,), jnp.int32),
            pltpu.VMEM((BG, VD), jnp.float32),
            pltpu.VMEM((BG, VD), jnp.float32),
            pltpu.SemaphoreType.DMA,
            pltpu.SemaphoreType.DMA,
        ],
    )
    return f(x1, idx_i, idx_j)


EPC = E // 2
EPT = EPC // 16
NZR = N // 16
BVN = 200


def _seg_n_body(msg_hbm, ii_hbm, agg_hbm, idx_v, vals_v, zbuf_v, acc_sh,
                sem1):
    c = lax.axis_index("c")
    s = lax.axis_index("s")
    zero16 = jnp.zeros((16,), jnp.float32)

    def zrow(r, carry):
        zbuf_v[r, pl.ds(0, 16)] = zero16
        zbuf_v[r, pl.ds(16, 16)] = zero16
        return carry

    lax.fori_loop(0, BVN, zrow, 0)
    zb = s * NZR
    for j in range(NZR // BVN):
        pltpu.sync_copy(zbuf_v, acc_sh.at[pl.ds(zb + j * BVN, BVN)])
    remz = NZR % BVN
    pltpu.sync_copy(zbuf_v.at[pl.ds(0, remz)],
                    acc_sh.at[pl.ds(zb + NZR - remz, remz)])
    plsc.subcore_barrier()

    tbase = c * EPC + s * EPT

    def rnd(r, carry):
        off = tbase + r * BVN
        pltpu.sync_copy(ii_hbm.at[pl.ds(off, BVN)], idx_v)
        pltpu.sync_copy(msg_hbm.at[pl.ds(off, BVN)], vals_v)
        pltpu.sync_copy(vals_v, acc_sh.at[idx_v], add=True)
        return carry

    lax.fori_loop(0, EPT // BVN, rnd, 0)
    plsc.subcore_barrier()

    for j in range(NZR // BVN):
        pltpu.sync_copy(acc_sh.at[pl.ds(zb + j * BVN, BVN)], vals_v)
        pltpu.sync_copy(vals_v, agg_hbm.at[c, pl.ds(zb + j * BVN, BVN)])
    pltpu.sync_copy(acc_sh.at[pl.ds(zb + NZR - remz, remz)],
                    vals_v.at[pl.ds(0, remz)])
    pltpu.sync_copy(vals_v.at[pl.ds(0, remz)],
                    agg_hbm.at[c, pl.ds(zb + NZR - remz, remz)])


def _seg_n_sc(msg, idx_i):
    mesh = plsc.VectorSubcoreMesh(core_axis_name="c", subcore_axis_name="s")
    f = pl.kernel(
        _seg_n_body,
        out_type=jax.ShapeDtypeStruct((2, N, VD), jnp.float32),
        mesh=mesh,
        compiler_params=pltpu.CompilerParams(needs_layout_passes=False,
                                             use_tc_tiling_on_sc=False),
        scratch_types=[
            pltpu.VMEM((BVN,), jnp.int32),
            pltpu.VMEM((BVN, VD), jnp.float32),
            pltpu.VMEM((BVN, VD), jnp.float32),
            pltpu.VMEM_SHARED((N, VD), jnp.float32),
            pltpu.SemaphoreType.DMA,
        ],
    )
    return f(msg, idx_i)


def kernel(x, cji, valence_mask, cutoff_w, rb, shb, idx_i, idx_j, tri_idx_k,
           edge_idx_kj, edge_idx_ji, W_node, b_node, W_c1, W_c2, W_t1, b_t1,
           W_t2, b_t2, W_basis, W_n1, b_n1, W_n2, b_n2, W_out):
    order = jnp.arange(VD, dtype=jnp.int32).reshape(VD // 2, 2).T.reshape(VD)
    wn_p = jnp.concatenate([W_node[:, :VD], W_node[:, VD:][:, order]], axis=1)
    bn_p = jnp.concatenate([b_node[:VD], b_node[VD:][order]])
    x1, xks = _prenode(x, wn_p, bn_p)
    cw, s = _cmlp(cji, rb, cutoff_w, W_c1, W_c2)

    tbw_t = _tri_sc(cw, shb.reshape(T * NORB), edge_idx_kj, tri_idx_k, xks)
    agg_e = _seg_e_sc(tbw_t, edge_idx_ji)

    nf_a, nf_b = _pair_sc(x1, idx_i, idx_j)

    msg = _post(agg_e, s, nf_a, nf_b, W_t1[order], b_t1, W_t2, b_t2,
                W_basis, W_n1, b_n1, W_n2, b_n2)

    agg_n2 = _seg_n_sc(msg, idx_i)
    return _final(x, agg_n2, W_out)
